# Initial kernel scaffold; baseline (speedup 1.0000x reference)
import jax
import jax.numpy as jnp
from jax import lax
from jax.experimental import pallas as pl
from jax.experimental.pallas import tpu as pltpu
from jax.experimental.pallas import tpu_sc as plsc

NUM_ITEMS = 1000000
BATCH = 16384
HALF = 524288
TPAD = 524416
ZCHUNK = TPAD // 16
EPB = BATCH // 16
DUMMY = HALF
NVEC = EPB // 16


def _sc_body(rating_h, item_h, titem_h, zeros_h, bt_h, ct_h, fb_h,
             sh_base, sh_cnt, item_v, rat_v, titem_v, idx_s, tidx_s,
             val_s, cnt_s, gb_s, gc_s, gbt_s, gct_s, bt_buf, ct_buf,
             fb_buf):
    cid = lax.axis_index("c")
    sid = lax.axis_index("s")
    lo = cid * HALF

    z0 = sid * ZCHUNK
    pltpu.sync_copy(zeros_h.at[pl.ds(z0, ZCHUNK)],
                    sh_base.at[pl.ds(z0, ZCHUNK)])
    pltpu.sync_copy(zeros_h.at[pl.ds(z0, ZCHUNK)],
                    sh_cnt.at[pl.ds(z0, ZCHUNK)])

    e0 = sid * EPB
    pltpu.sync_copy(item_h.at[pl.ds(e0, EPB)], item_v)
    pltpu.sync_copy(rating_h.at[pl.ds(e0, EPB)], rat_v)
    pltpu.sync_copy(titem_h.at[pl.ds(e0, EPB)], titem_v)

    def prep(i, _):
        j = i // 8
        c = (i % 8) * 16
        it = item_v[pl.ds(i * 16, 16)]
        rt = rat_v[pl.ds(i * 16, 16)]
        tt = titem_v[pl.ds(i * 16, 16)]
        inr = (it >= lo) & (it < lo + HALF)
        idx_s[j, pl.ds(c, 16)] = jnp.where(inr, it - lo, DUMMY)
        val_s[j, pl.ds(c, 16)] = jnp.where(inr, rt, 0.0)
        cnt_s[j, pl.ds(c, 16)] = jnp.where(inr & (rt > 0.0), 1.0, 0.0)
        tinr = (tt >= lo) & (tt < lo + HALF)
        tidx_s[j, pl.ds(c, 16)] = jnp.where(tinr, tt - lo, DUMMY)
        return 0

    lax.fori_loop(0, NVEC, prep, 0)

    plsc.subcore_barrier()

    for j in range(8):
        pltpu.sync_copy(val_s.at[j], sh_base.at[idx_s.at[j]], add=True)
        pltpu.sync_copy(cnt_s.at[j], sh_cnt.at[idx_s.at[j]], add=True)

    plsc.subcore_barrier()

    for j in range(8):
        pltpu.sync_copy(sh_base.at[idx_s.at[j]], gb_s.at[j])
        pltpu.sync_copy(sh_cnt.at[idx_s.at[j]], gc_s.at[j])
        pltpu.sync_copy(sh_base.at[tidx_s.at[j]], gbt_s.at[j])
        pltpu.sync_copy(sh_cnt.at[tidx_s.at[j]], gct_s.at[j])

    def comp(i, carry):
        num, nnz = carry
        j = i // 8
        c = (i % 8) * 16
        sel = cnt_s[j, pl.ds(c, 16)]
        gb = gb_s[j, pl.ds(c, 16)]
        gc = gc_s[j, pl.ds(c, 16)]
        ceff = jnp.where(sel > 0.0, gc, 1.0)
        num = num + sel * gb / (ceff * ceff)
        nnz = nnz + sel / ceff
        tt = titem_v[pl.ds(i * 16, 16)]
        tinr = (tt >= lo) & (tt < lo + HALF)
        bt_buf[pl.ds(i * 16, 16)] = jnp.where(
            tinr, gbt_s[j, pl.ds(c, 16)], 0.0)
        ct_buf[pl.ds(i * 16, 16)] = jnp.where(
            tinr, gct_s[j, pl.ds(c, 16)], 0.0)
        return num, nnz

    zero16 = jnp.zeros((16,), jnp.float32)
    num, nnz = lax.fori_loop(0, NVEC, comp, (zero16, zero16))
    fb_buf[0, :] = num
    fb_buf[1, :] = nnz

    pltpu.sync_copy(bt_buf, bt_h.at[cid, pl.ds(e0, EPB)])
    pltpu.sync_copy(ct_buf, ct_h.at[cid, pl.ds(e0, EPB)])
    pltpu.sync_copy(fb_buf, fb_h.at[cid, sid])


def _tc_epilogue(bt_ref, ct_ref, num_ref, nnz_ref, tr_ref, pred_ref,
                 loss_ref):
    bt = bt_ref[0:128, :] + bt_ref[128:256, :]
    ct = ct_ref[0:128, :] + ct_ref[128:256, :]
    num = jnp.sum(num_ref[...])
    nnz = jnp.sum(nnz_ref[...])
    fb = num / jnp.maximum(nnz, 1.0)
    pred = jnp.where(ct == 0.0, fb, bt / (ct + 1e-10))
    pred_ref[...] = pred
    err = pred - tr_ref[...]
    loss_ref[0, 0] = jnp.sum(err * err) * (1.0 / BATCH)


def kernel(rating, item, target_rating, target_item, base, count):
    item = item.astype(jnp.int32)
    target_item = target_item.astype(jnp.int32)
    zeros = jnp.zeros((TPAD,), jnp.float32)

    sc_call = pl.kernel(
        _sc_body,
        out_type=[
            jax.ShapeDtypeStruct((2, BATCH), jnp.float32),
            jax.ShapeDtypeStruct((2, BATCH), jnp.float32),
            jax.ShapeDtypeStruct((2, 16, 2, 16), jnp.float32),
        ],
        scratch_types=[
            pltpu.VMEM_SHARED((TPAD,), jnp.float32),
            pltpu.VMEM_SHARED((TPAD,), jnp.float32),
            pltpu.VMEM((EPB,), jnp.int32),
            pltpu.VMEM((EPB,), jnp.float32),
            pltpu.VMEM((EPB,), jnp.int32),
            pltpu.VMEM((8, 128), jnp.int32),
            pltpu.VMEM((8, 128), jnp.int32),
            pltpu.VMEM((8, 128), jnp.float32),
            pltpu.VMEM((8, 128), jnp.float32),
            pltpu.VMEM((8, 128), jnp.float32),
            pltpu.VMEM((8, 128), jnp.float32),
            pltpu.VMEM((8, 128), jnp.float32),
            pltpu.VMEM((8, 128), jnp.float32),
            pltpu.VMEM((EPB,), jnp.float32),
            pltpu.VMEM((EPB,), jnp.float32),
            pltpu.VMEM((2, 16), jnp.float32),
        ],
        mesh=plsc.VectorSubcoreMesh(core_axis_name="c", subcore_axis_name="s"),
    )
    bt_part, ct_part, fb_part = sc_call(rating, item, target_item, zeros)

    bt2 = bt_part.reshape(256, 128)
    ct2 = ct_part.reshape(256, 128)
    fb2 = fb_part.reshape(64, 2, 16)
    num_mat = fb2[:, 0, :].reshape(8, 128)
    nnz_mat = fb2[:, 1, :].reshape(8, 128)
    tr2 = target_rating.reshape(128, 128)

    pred2, loss2 = pl.pallas_call(
        _tc_epilogue,
        out_shape=[
            jax.ShapeDtypeStruct((128, 128), jnp.float32),
            jax.ShapeDtypeStruct((1, 1), jnp.float32),
        ],
    )(bt2, ct2, num_mat, nnz_mat, tr2)

    return pred2.reshape(BATCH), loss2[0, 0]


# speedup vs baseline: 2.5957x; 2.5957x over previous
import jax
import jax.numpy as jnp
from jax import lax
from jax.experimental import pallas as pl
from jax.experimental.pallas import tpu as pltpu
from jax.experimental.pallas import tpu_sc as plsc

NUM_ITEMS = 1000000
BATCH = 16384
HALF = 524288
TPAD = 524416
ZCHUNK = TPAD // 16
EPB = BATCH // 16
DUMMY = HALF
NVEC = EPB // 16


def _sc_body(rating_h, item_h, titem_h, zeros_h, bt_h, ct_h, fb_h,
             sh_base, sh_cnt, zbuf, item_v, rat_v, titem_v, idx_s, tidx_s,
             val_s, cnt_s, gb_s, gc_s, gbt_s, gct_s, bt_buf, ct_buf,
             fb_buf):
    cid = lax.axis_index("c")
    sid = lax.axis_index("s")
    lo = cid * HALF

    z0 = sid * ZCHUNK
    pltpu.sync_copy(zeros_h, zbuf)
    pltpu.sync_copy(zbuf, sh_base.at[pl.ds(z0, ZCHUNK)])
    pltpu.sync_copy(zbuf, sh_cnt.at[pl.ds(z0, ZCHUNK)])

    e0 = sid * EPB
    pltpu.sync_copy(item_h.at[pl.ds(e0, EPB)], item_v)
    pltpu.sync_copy(rating_h.at[pl.ds(e0, EPB)], rat_v)
    pltpu.sync_copy(titem_h.at[pl.ds(e0, EPB)], titem_v)

    def prep(i, _):
        j = i // 8
        c = (i % 8) * 16
        it = item_v[pl.ds(i * 16, 16)]
        rt = rat_v[pl.ds(i * 16, 16)]
        tt = titem_v[pl.ds(i * 16, 16)]
        inr = (it >= lo) & (it < lo + HALF)
        idx_s[j, pl.ds(c, 16)] = jnp.where(inr, it - lo, DUMMY)
        val_s[j, pl.ds(c, 16)] = jnp.where(inr, rt, 0.0)
        cnt_s[j, pl.ds(c, 16)] = jnp.where(inr & (rt > 0.0), 1.0, 0.0)
        tinr = (tt >= lo) & (tt < lo + HALF)
        tidx_s[j, pl.ds(c, 16)] = jnp.where(tinr, tt - lo, DUMMY)
        return 0

    lax.fori_loop(0, NVEC, prep, 0)

    plsc.subcore_barrier()

    for j in range(8):
        pltpu.sync_copy(val_s.at[j], sh_base.at[idx_s.at[j]], add=True)
        pltpu.sync_copy(cnt_s.at[j], sh_cnt.at[idx_s.at[j]], add=True)

    plsc.subcore_barrier()

    for j in range(8):
        pltpu.sync_copy(sh_base.at[idx_s.at[j]], gb_s.at[j])
        pltpu.sync_copy(sh_cnt.at[idx_s.at[j]], gc_s.at[j])
        pltpu.sync_copy(sh_base.at[tidx_s.at[j]], gbt_s.at[j])
        pltpu.sync_copy(sh_cnt.at[tidx_s.at[j]], gct_s.at[j])

    def comp(i, carry):
        num, nnz = carry
        j = i // 8
        c = (i % 8) * 16
        sel = cnt_s[j, pl.ds(c, 16)]
        gb = gb_s[j, pl.ds(c, 16)]
        gc = gc_s[j, pl.ds(c, 16)]
        ceff = jnp.where(sel > 0.0, gc, 1.0)
        num = num + sel * gb / (ceff * ceff)
        nnz = nnz + sel / ceff
        tt = titem_v[pl.ds(i * 16, 16)]
        tinr = (tt >= lo) & (tt < lo + HALF)
        bt_buf[pl.ds(i * 16, 16)] = jnp.where(
            tinr, gbt_s[j, pl.ds(c, 16)], 0.0)
        ct_buf[pl.ds(i * 16, 16)] = jnp.where(
            tinr, gct_s[j, pl.ds(c, 16)], 0.0)
        return num, nnz

    zero16 = jnp.zeros((16,), jnp.float32)
    num, nnz = lax.fori_loop(0, NVEC, comp, (zero16, zero16))
    fb_buf[0, :] = num
    fb_buf[1, :] = nnz

    pltpu.sync_copy(bt_buf, bt_h.at[cid, pl.ds(e0, EPB)])
    pltpu.sync_copy(ct_buf, ct_h.at[cid, pl.ds(e0, EPB)])
    pltpu.sync_copy(fb_buf, fb_h.at[cid, sid])


def _tc_epilogue(bt_ref, ct_ref, num_ref, nnz_ref, tr_ref, pred_ref,
                 loss_ref):
    bt = bt_ref[0:128, :] + bt_ref[128:256, :]
    ct = ct_ref[0:128, :] + ct_ref[128:256, :]
    num = jnp.sum(num_ref[...])
    nnz = jnp.sum(nnz_ref[...])
    fb = num / jnp.maximum(nnz, 1.0)
    pred = jnp.where(ct == 0.0, fb, bt / (ct + 1e-10))
    pred_ref[...] = pred
    err = pred - tr_ref[...]
    loss_ref[...] = (jnp.sum(err * err) * (1.0 / BATCH)).reshape(1, 1)


def kernel(rating, item, target_rating, target_item, base, count):
    item = item.astype(jnp.int32)
    target_item = target_item.astype(jnp.int32)
    zeros = jnp.zeros((ZCHUNK,), jnp.float32)

    sc_call = pl.kernel(
        _sc_body,
        out_type=[
            jax.ShapeDtypeStruct((2, BATCH), jnp.float32),
            jax.ShapeDtypeStruct((2, BATCH), jnp.float32),
            jax.ShapeDtypeStruct((2, 16, 2, 16), jnp.float32),
        ],
        scratch_types=[
            pltpu.VMEM_SHARED((TPAD,), jnp.float32),
            pltpu.VMEM_SHARED((TPAD,), jnp.float32),
            pltpu.VMEM((ZCHUNK,), jnp.float32),
            pltpu.VMEM((EPB,), jnp.int32),
            pltpu.VMEM((EPB,), jnp.float32),
            pltpu.VMEM((EPB,), jnp.int32),
            pltpu.VMEM((8, 128), jnp.int32),
            pltpu.VMEM((8, 128), jnp.int32),
            pltpu.VMEM((8, 128), jnp.float32),
            pltpu.VMEM((8, 128), jnp.float32),
            pltpu.VMEM((8, 128), jnp.float32),
            pltpu.VMEM((8, 128), jnp.float32),
            pltpu.VMEM((8, 128), jnp.float32),
            pltpu.VMEM((8, 128), jnp.float32),
            pltpu.VMEM((EPB,), jnp.float32),
            pltpu.VMEM((EPB,), jnp.float32),
            pltpu.VMEM((2, 16), jnp.float32),
        ],
        mesh=plsc.VectorSubcoreMesh(core_axis_name="c", subcore_axis_name="s"),
    )
    bt_part, ct_part, fb_part = sc_call(rating, item, target_item, zeros)

    bt2 = bt_part.reshape(256, 128)
    ct2 = ct_part.reshape(256, 128)
    fb2 = fb_part.reshape(32, 2, 16)
    num_mat = fb2[:, 0, :].reshape(4, 128)
    nnz_mat = fb2[:, 1, :].reshape(4, 128)
    tr2 = target_rating.reshape(128, 128)

    pred2, loss2 = pl.pallas_call(
        _tc_epilogue,
        out_shape=[
            jax.ShapeDtypeStruct((128, 128), jnp.float32),
            jax.ShapeDtypeStruct((1, 1), jnp.float32),
        ],
    )(bt2, ct2, num_mat, nnz_mat, tr2)

    return pred2.reshape(BATCH), loss2[0, 0]
